# Initial kernel scaffold; baseline (speedup 1.0000x reference)
import jax
import jax.numpy as jnp
from jax.experimental import pallas as pl


def kernel(inputs, context, embeddings):
    raise NotImplementedError("write your pallas kernel here")


# speedup vs baseline: 1.7338x; 1.7338x over previous
import functools

import jax
import jax.numpy as jnp
from jax.experimental import pallas as pl
from jax.experimental.pallas import tpu as pltpu

_N_E = 1024
_D = 64
_COST = 0.25


def _vq_body(n_rows, tile, grid,
             x_ref, e_ref,
             dist_ref, idx_ref, enc_ref, q_ref, loss_ref, perp_ref,
             cnt_acc, loss_acc):
    i = pl.program_id(0)

    @pl.when(i == 0)
    def _init():
        cnt_acc[...] = jnp.zeros_like(cnt_acc)
        loss_acc[0] = 0.0

    x = x_ref[...]
    e = e_ref[...]
    xsq = jnp.sum(x * x, axis=1, keepdims=True)
    esq = jnp.sum(e * e, axis=0, keepdims=True)
    mm = jax.lax.dot_general(x, e, (((1,), (0,)), ((), ())),
                             precision=jax.lax.Precision.DEFAULT)
    dist = (xsq - 2.0 * mm) + esq
    dist_ref[...] = dist

    m = jnp.min(dist, axis=1, keepdims=True)
    col = jax.lax.broadcasted_iota(jnp.int32, (tile, _N_E), 1)
    hit = dist == m
    idx = jnp.min(jnp.where(hit, col, _N_E), axis=1, keepdims=True)
    idx_ref[...] = idx

    enc = jnp.where(col == idx, 1.0, 0.0).astype(jnp.float32)
    enc_ref[...] = enc

    q = jax.lax.dot_general(enc, e, (((1,), (1,)), ((), ())),
                            precision=jax.lax.Precision.HIGHEST)
    q_ref[...] = x + (q - x)

    cnt_acc[...] += jnp.sum(enc, axis=0, keepdims=True)
    loss_acc[0] += jnp.sum(m)

    @pl.when(i == grid - 1)
    def _fin():
        total = loss_acc[0]
        loss_ref[0, 0] = (1.0 + _COST) * (total / float(n_rows * _D))
        avg = cnt_acc[...] * (1.0 / float(n_rows))
        ent = jnp.sum(avg * jnp.log(avg + 1e-10))
        perp_ref[0, 0] = jnp.exp(-ent)


def kernel(inputs, context, embeddings):
    del context
    n_rows = inputs.shape[0] * inputs.shape[1]
    tile = 512
    grid = n_rows // tile
    x = jnp.reshape(inputs, (n_rows, _D))

    out_shapes = (
        jax.ShapeDtypeStruct((n_rows, _N_E), jnp.float32),
        jax.ShapeDtypeStruct((n_rows, 1), jnp.int32),
        jax.ShapeDtypeStruct((n_rows, _N_E), jnp.float32),
        jax.ShapeDtypeStruct((n_rows, _D), jnp.float32),
        jax.ShapeDtypeStruct((1, 1), jnp.float32),
        jax.ShapeDtypeStruct((1, 1), jnp.float32),
    )
    dist, idx, enc, q, loss, perp = pl.pallas_call(
        functools.partial(_vq_body, n_rows, tile, grid),
        grid=(grid,),
        in_specs=[
            pl.BlockSpec((tile, _D), lambda i: (i, 0)),
            pl.BlockSpec((_D, _N_E), lambda i: (0, 0)),
        ],
        out_specs=[
            pl.BlockSpec((tile, _N_E), lambda i: (i, 0)),
            pl.BlockSpec((tile, 1), lambda i: (i, 0)),
            pl.BlockSpec((tile, _N_E), lambda i: (i, 0)),
            pl.BlockSpec((tile, _D), lambda i: (i, 0)),
            pl.BlockSpec(memory_space=pltpu.SMEM),
            pl.BlockSpec(memory_space=pltpu.SMEM),
        ],
        out_shape=out_shapes,
        scratch_shapes=[
            pltpu.VMEM((1, _N_E), jnp.float32),
            pltpu.SMEM((1,), jnp.float32),
        ],
    )(x, embeddings)

    quantized = jnp.reshape(q, inputs.shape)
    encoding_indices = jnp.reshape(idx, inputs.shape[:-1])
    return (quantized, jnp.reshape(loss, ()), jnp.reshape(perp, ()),
            enc, encoding_indices, dist)


# speedup vs baseline: 1.9653x; 1.1336x over previous
import functools

import jax
import jax.numpy as jnp
from jax import lax
from jax.experimental import pallas as pl
from jax.experimental.pallas import tpu as pltpu
from jax.experimental.pallas import tpu_sc as plsc

_N_E = 1024
_D = 64
_COST = 0.25


def _vq_body(n_rows, tile, grid,
             x_ref, e_ref,
             dist_ref, idx_ref, enc_ref, loss_ref, perp_ref,
             cnt_acc, loss_acc):
    i = pl.program_id(0)

    @pl.when(i == 0)
    def _init():
        cnt_acc[...] = jnp.zeros_like(cnt_acc)
        loss_acc[0] = 0.0

    x = x_ref[0]
    e = e_ref[...]
    xsq = jnp.sum(x * x, axis=1, keepdims=True)
    esq = jnp.sum(e * e, axis=0, keepdims=True)
    mm = jax.lax.dot_general(x, e, (((1,), (0,)), ((), ())),
                             precision=jax.lax.Precision.DEFAULT)
    dist = (xsq - 2.0 * mm) + esq
    dist_ref[...] = dist

    m = jnp.min(dist, axis=1, keepdims=True)
    col = jax.lax.broadcasted_iota(jnp.int32, (tile, _N_E), 1)
    idx = jnp.min(jnp.where(dist == m, col, _N_E), axis=1, keepdims=True)
    idx_ref[0] = jnp.reshape(idx, (1, tile))

    enc = jnp.where(col == idx, 1.0, 0.0).astype(jnp.float32)
    enc_ref[...] = enc

    cnt_acc[...] += jnp.sum(enc, axis=0, keepdims=True)
    loss_acc[0] += jnp.sum(m)

    @pl.when(i == grid - 1)
    def _fin():
        total = loss_acc[0]
        loss_ref[0, 0] = (1.0 + _COST) * (total / float(n_rows * _D))
        avg = cnt_acc[...] * (1.0 / float(n_rows))
        ent = jnp.sum(avg * jnp.log(avg + 1e-10))
        perp_ref[0, 0] = jnp.exp(-ent)


def _make_sc_gather(n_rows):
    info = plsc.get_sparse_core_info()
    nc, ns = info.num_cores, info.num_subcores
    nw = nc * ns
    bpw = n_rows // nw
    chunks = [128] * (bpw // 128)
    if bpw % 128:
        chunks.append(bpw % 128)
    mesh = plsc.VectorSubcoreMesh(core_axis_name="c", subcore_axis_name="s")

    @functools.partial(
        pl.kernel, mesh=mesh,
        out_type=jax.ShapeDtypeStruct((n_rows, 128), jnp.float32),
        scratch_types=[
            pltpu.VMEM((bpw,), jnp.int32),
            pltpu.VMEM((bpw, 128), jnp.float32),
            pltpu.SemaphoreType.DMA,
        ],
    )
    def sc_gather(table_hbm, idx_hbm, out_hbm, idx_v, rows_v, sem):
        wid = lax.axis_index("s") * nc + lax.axis_index("c")
        base = wid * bpw
        pltpu.sync_copy(idx_hbm.at[pl.ds(base, bpw)], idx_v)
        copies = []
        off = 0
        for n in chunks:
            copies.append(pltpu.async_copy(
                table_hbm.at[idx_v.at[pl.ds(off, n)]],
                rows_v.at[pl.ds(off, n)], sem))
            off += n
        for c in copies:
            c.wait()
        pltpu.sync_copy(rows_v, out_hbm.at[pl.ds(base, bpw)])

    return sc_gather


def kernel(inputs, context, embeddings):
    del context
    b, s = inputs.shape[0], inputs.shape[1]
    n_rows = b * s
    tile = s
    grid = b

    out_shapes = (
        jax.ShapeDtypeStruct((n_rows, _N_E), jnp.float32),
        jax.ShapeDtypeStruct((b, 1, s), jnp.int32),
        jax.ShapeDtypeStruct((n_rows, _N_E), jnp.float32),
        jax.ShapeDtypeStruct((1, 1), jnp.float32),
        jax.ShapeDtypeStruct((1, 1), jnp.float32),
    )
    dist, idx3, enc, loss, perp = pl.pallas_call(
        functools.partial(_vq_body, n_rows, tile, grid),
        grid=(grid,),
        in_specs=[
            pl.BlockSpec((1, s, _D), lambda i: (i, 0, 0)),
            pl.BlockSpec((_D, _N_E), lambda i: (0, 0)),
        ],
        out_specs=[
            pl.BlockSpec((tile, _N_E), lambda i: (i, 0)),
            pl.BlockSpec((1, 1, s), lambda i: (i, 0, 0)),
            pl.BlockSpec((tile, _N_E), lambda i: (i, 0)),
            pl.BlockSpec(memory_space=pltpu.SMEM),
            pl.BlockSpec(memory_space=pltpu.SMEM),
        ],
        out_shape=out_shapes,
        scratch_shapes=[
            pltpu.VMEM((1, _N_E), jnp.float32),
            pltpu.SMEM((1,), jnp.float32),
        ],
    )(inputs, embeddings)

    idx_flat = jnp.reshape(idx3, (n_rows,))
    table = jnp.swapaxes(embeddings, 0, 1)
    table128 = jnp.pad(table, ((0, 0), (0, 128 - _D)))
    q128 = _make_sc_gather(n_rows)(table128, idx_flat)

    quantized = jnp.reshape(q128[:, :_D], inputs.shape)
    encoding_indices = jnp.reshape(idx3, (b, s))
    return (quantized, jnp.reshape(loss, ()), jnp.reshape(perp, ()),
            enc, encoding_indices, dist)


# speedup vs baseline: 2.0348x; 1.0353x over previous
import functools

import jax
import jax.numpy as jnp
from jax import lax
from jax.experimental import pallas as pl
from jax.experimental.pallas import tpu as pltpu
from jax.experimental.pallas import tpu_sc as plsc

_N_E = 1024
_D = 64
_COST = 0.25


def _vq_body(n_rows, tile, grid,
             x_hbm, e_ref,
             dist_ref, idx_ref, enc_ref, loss_ref, perp_ref,
             cnt_acc, loss_acc, x_buf, x_sem):
    i = pl.program_id(0)

    @pl.when(i == 0)
    def _init():
        cnt_acc[...] = jnp.zeros_like(cnt_acc)
        loss_acc[0] = 0.0
        pltpu.make_async_copy(x_hbm.at[0], x_buf.at[0], x_sem.at[0]).start()

    @pl.when(i + 1 < grid)
    def _next():
        pltpu.make_async_copy(x_hbm.at[i + 1], x_buf.at[(i + 1) % 2],
                              x_sem.at[(i + 1) % 2]).start()

    slot = i % 2
    pltpu.make_async_copy(x_hbm.at[i], x_buf.at[slot], x_sem.at[slot]).wait()
    x = x_buf[slot]
    e = e_ref[...]
    xsq = jnp.sum(x * x, axis=1, keepdims=True)
    esq = jnp.sum(e * e, axis=0, keepdims=True)
    mm = jax.lax.dot_general(x, e, (((1,), (0,)), ((), ())),
                             precision=jax.lax.Precision.DEFAULT)
    dist = (xsq - 2.0 * mm) + esq
    dist_ref[...] = dist

    m = jnp.min(dist, axis=1, keepdims=True)
    colf = jax.lax.broadcasted_iota(
        jnp.int32, (tile, _N_E), 1).astype(jnp.float32)
    idxf = jnp.min(jnp.where(dist == m, colf, 2048.0), axis=1, keepdims=True)
    idx_ref[0] = jnp.reshape(idxf.astype(jnp.int32), (1, tile))

    enc = jnp.where(colf == idxf, 1.0, 0.0).astype(jnp.float32)
    enc_ref[...] = enc

    cnt_acc[...] += jnp.sum(enc, axis=0, keepdims=True)
    loss_acc[0] += jnp.sum(m)

    @pl.when(i == grid - 1)
    def _fin():
        total = loss_acc[0]
        loss_ref[0, 0] = (1.0 + _COST) * (total / float(n_rows * _D))
        avg = cnt_acc[...] * (1.0 / float(n_rows))
        ent = jnp.sum(avg * jnp.log(avg + 1e-10))
        perp_ref[0, 0] = jnp.exp(-ent)


def _make_sc_gather(n_rows):
    info = plsc.get_sparse_core_info()
    nc, ns = info.num_cores, info.num_subcores
    nw = nc * ns
    bpw = n_rows // nw
    chunks = [128] * (bpw // 128)
    if bpw % 128:
        chunks.append(bpw % 128)
    mesh = plsc.VectorSubcoreMesh(core_axis_name="c", subcore_axis_name="s")

    @functools.partial(
        pl.kernel, mesh=mesh,
        out_type=jax.ShapeDtypeStruct((n_rows, 128), jnp.float32),
        scratch_types=[
            pltpu.VMEM((bpw,), jnp.int32),
            pltpu.VMEM((bpw, 128), jnp.float32),
            pltpu.SemaphoreType.DMA,
        ],
    )
    def sc_gather(table_hbm, idx_hbm, out_hbm, idx_v, rows_v, sem):
        wid = lax.axis_index("s") * nc + lax.axis_index("c")
        base = wid * bpw
        pltpu.sync_copy(idx_hbm.at[pl.ds(base, bpw)], idx_v)
        copies = []
        off = 0
        for n in chunks:
            copies.append(pltpu.async_copy(
                table_hbm.at[idx_v.at[pl.ds(off, n)]],
                rows_v.at[pl.ds(off, n)], sem))
            off += n
        for c in copies:
            c.wait()
        pltpu.sync_copy(rows_v, out_hbm.at[pl.ds(base, bpw)])

    return sc_gather


def kernel(inputs, context, embeddings):
    del context
    b, s = inputs.shape[0], inputs.shape[1]
    n_rows = b * s
    tile = s
    grid = b

    out_shapes = (
        jax.ShapeDtypeStruct((n_rows, _N_E), jnp.float32),
        jax.ShapeDtypeStruct((b, 1, s), jnp.int32),
        jax.ShapeDtypeStruct((n_rows, _N_E), jnp.float32),
        jax.ShapeDtypeStruct((1, 1), jnp.float32),
        jax.ShapeDtypeStruct((1, 1), jnp.float32),
    )
    dist, idx3, enc, loss, perp = pl.pallas_call(
        functools.partial(_vq_body, n_rows, tile, grid),
        grid=(grid,),
        in_specs=[
            pl.BlockSpec(memory_space=pltpu.MemorySpace.HBM),
            pl.BlockSpec((_D, _N_E), lambda i: (0, 0)),
        ],
        out_specs=[
            pl.BlockSpec((tile, _N_E), lambda i: (i, 0)),
            pl.BlockSpec((1, 1, s), lambda i: (i, 0, 0)),
            pl.BlockSpec((tile, _N_E), lambda i: (i, 0)),
            pl.BlockSpec(memory_space=pltpu.SMEM),
            pl.BlockSpec(memory_space=pltpu.SMEM),
        ],
        out_shape=out_shapes,
        scratch_shapes=[
            pltpu.VMEM((1, _N_E), jnp.float32),
            pltpu.SMEM((1,), jnp.float32),
            pltpu.VMEM((2, s, _D), jnp.float32),
            pltpu.SemaphoreType.DMA((2,)),
        ],
    )(inputs, embeddings)

    idx_flat = jnp.reshape(idx3, (n_rows,))
    table = jnp.swapaxes(embeddings, 0, 1)
    table128 = jnp.pad(table, ((0, 0), (0, 128 - _D)))
    q128 = _make_sc_gather(n_rows)(table128, idx_flat)

    quantized = jnp.reshape(q128[:, :_D], inputs.shape)
    encoding_indices = jnp.reshape(idx3, (b, s))
    return (quantized, jnp.reshape(loss, ()), jnp.reshape(perp, ()),
            enc, encoding_indices, dist)


# speedup vs baseline: 2.2053x; 1.0838x over previous
import functools

import jax
import jax.numpy as jnp
from jax import lax
from jax.experimental import pallas as pl
from jax.experimental.pallas import tpu as pltpu
from jax.experimental.pallas import tpu_sc as plsc

_N_E = 1024
_D = 64
_COST = 0.25


def _vq_body(n_rows, tile, grid,
             x_hbm, e_ref,
             dist_ref, idx_ref, enc_ref, loss_ref, perp_ref,
             cnt_acc, loss_acc, x_buf, x_sem):
    i = pl.program_id(0)

    @pl.when(i == 0)
    def _init():
        cnt_acc[...] = jnp.zeros_like(cnt_acc)
        loss_acc[0] = 0.0
        pltpu.make_async_copy(x_hbm.at[0], x_buf.at[0], x_sem.at[0]).start()

    @pl.when(i + 1 < grid)
    def _next():
        pltpu.make_async_copy(x_hbm.at[i + 1], x_buf.at[(i + 1) % 2],
                              x_sem.at[(i + 1) % 2]).start()

    slot = i % 2
    pltpu.make_async_copy(x_hbm.at[i], x_buf.at[slot], x_sem.at[slot]).wait()
    x = x_buf[slot]
    e = e_ref[...]
    xsq = jnp.sum(x * x, axis=1, keepdims=True)
    esq = jnp.sum(e * e, axis=0, keepdims=True)
    mm = jax.lax.dot_general(x, e, (((1,), (0,)), ((), ())),
                             precision=jax.lax.Precision.DEFAULT)
    dist = (xsq - 2.0 * mm) + esq
    dist_ref[...] = dist

    m = jnp.min(dist, axis=1, keepdims=True)
    colf = jax.lax.broadcasted_iota(
        jnp.int32, (tile, _N_E), 1).astype(jnp.float32)
    idxf = jnp.min(jnp.where(dist == m, colf, 2048.0), axis=1, keepdims=True)
    idx_ref[0] = jnp.reshape(idxf.astype(jnp.int32), (1, tile))

    enc = jnp.where(colf == idxf, 1.0, 0.0).astype(jnp.float32)
    enc_ref[...] = enc

    cnt_acc[...] += jnp.sum(enc, axis=0, keepdims=True)
    loss_acc[0] += jnp.sum(m)

    @pl.when(i == grid - 1)
    def _fin():
        total = loss_acc[0]
        loss_ref[0, 0] = (1.0 + _COST) * (total / float(n_rows * _D))
        avg = cnt_acc[...] * (1.0 / float(n_rows))
        ent = jnp.sum(avg * jnp.log(avg + 1e-10))
        perp_ref[0, 0] = jnp.exp(-ent)


def _make_sc_gather(n_rows):
    info = plsc.get_sparse_core_info()
    nc, ns = info.num_cores, info.num_subcores
    nw = nc * ns
    bpw = n_rows // nw
    chunks = [128] * (bpw // 128)
    if bpw % 128:
        chunks.append(bpw % 128)
    mesh = plsc.VectorSubcoreMesh(core_axis_name="c", subcore_axis_name="s")

    @functools.partial(
        pl.kernel, mesh=mesh,
        out_type=jax.ShapeDtypeStruct((n_rows, 128), jnp.float32),
        scratch_types=[
            pltpu.VMEM((bpw,), jnp.int32),
            pltpu.VMEM((bpw, 128), jnp.float32),
            pltpu.SemaphoreType.DMA,
        ],
    )
    def sc_gather(table_hbm, idx_hbm, out_hbm, idx_v, rows_v, sem):
        wid = lax.axis_index("s") * nc + lax.axis_index("c")
        base = wid * bpw
        pltpu.sync_copy(idx_hbm.at[pl.ds(base, bpw)], idx_v)
        copies = []
        off = 0
        for n in chunks:
            copies.append(pltpu.async_copy(
                table_hbm.at[idx_v.at[pl.ds(off, n)]],
                rows_v.at[pl.ds(off, n)], sem))
            off += n
        for c in copies:
            c.wait()
        pltpu.sync_copy(rows_v, out_hbm.at[pl.ds(base, bpw)])

    return sc_gather


def kernel(inputs, context, embeddings):
    del context
    b, s = inputs.shape[0], inputs.shape[1]
    n_rows = b * s
    tile = 2 * s
    grid = n_rows // tile
    x3 = jnp.reshape(inputs, (grid, tile, _D))

    out_shapes = (
        jax.ShapeDtypeStruct((n_rows, _N_E), jnp.float32),
        jax.ShapeDtypeStruct((grid, 1, tile), jnp.int32),
        jax.ShapeDtypeStruct((n_rows, _N_E), jnp.float32),
        jax.ShapeDtypeStruct((1, 1), jnp.float32),
        jax.ShapeDtypeStruct((1, 1), jnp.float32),
    )
    dist, idx3, enc, loss, perp = pl.pallas_call(
        functools.partial(_vq_body, n_rows, tile, grid),
        grid=(grid,),
        in_specs=[
            pl.BlockSpec(memory_space=pltpu.MemorySpace.HBM),
            pl.BlockSpec((_D, _N_E), lambda i: (0, 0)),
        ],
        out_specs=[
            pl.BlockSpec((tile, _N_E), lambda i: (i, 0)),
            pl.BlockSpec((1, 1, tile), lambda i: (i, 0, 0)),
            pl.BlockSpec((tile, _N_E), lambda i: (i, 0)),
            pl.BlockSpec(memory_space=pltpu.SMEM),
            pl.BlockSpec(memory_space=pltpu.SMEM),
        ],
        out_shape=out_shapes,
        scratch_shapes=[
            pltpu.VMEM((1, _N_E), jnp.float32),
            pltpu.SMEM((1,), jnp.float32),
            pltpu.VMEM((2, tile, _D), jnp.float32),
            pltpu.SemaphoreType.DMA((2,)),
        ],
    )(x3, embeddings)

    idx_flat = jnp.reshape(idx3, (n_rows,))
    table = jnp.swapaxes(embeddings, 0, 1)
    table128 = jnp.pad(table, ((0, 0), (0, 128 - _D)))
    q128 = _make_sc_gather(n_rows)(table128, idx_flat)

    quantized = jnp.reshape(q128[:, :_D], inputs.shape)
    encoding_indices = jnp.reshape(idx3, (b, s))
    return (quantized, jnp.reshape(loss, ()), jnp.reshape(perp, ()),
            enc, encoding_indices, dist)
